# baseline (speedup 1.0000x reference)
import functools

import jax
import jax.numpy as jnp
from jax import lax
from jax.experimental import pallas as pl
from jax.experimental.pallas import tpu as pltpu
from jax.experimental.pallas import tpu_sc as plsc

_BATCH = 16384
_EMBED = 32
_NC = 2
_NS = 16
_NW = _NC * _NS
_CHUNK = _BATCH // _NW
_L = 16
_IDXC = 128
_NIDX = _CHUNK // _IDXC
_GROUPS = _CHUNK // _L


def _sc_body(user_idx_hbm, item_idx_hbm, user_table_hbm, item_table_hbm,
             w_hbm, b_hbm, out_hbm,
             idx_u, idx_i, u_rows, i_rows, w_v, b_v, out_v, sem):
    wid = lax.axis_index("s") * _NC + lax.axis_index("c")
    base = wid * _CHUNK

    pltpu.sync_copy(user_idx_hbm.at[pl.ds(wid * _NIDX, _NIDX)], idx_u)
    pltpu.sync_copy(item_idx_hbm.at[pl.ds(wid * _NIDX, _NIDX)], idx_i)
    pltpu.sync_copy(w_hbm, w_v)
    pltpu.sync_copy(b_hbm, b_v)

    copies = []
    for k in range(_NIDX):
        copies.append(pltpu.async_copy(
            user_table_hbm.at[idx_u.at[k]],
            u_rows.at[pl.ds(k * _IDXC, _IDXC)], sem))
        copies.append(pltpu.async_copy(
            item_table_hbm.at[idx_i.at[k]],
            i_rows.at[pl.ds(k * _IDXC, _IDXC)], sem))
    for c in copies:
        c.wait()

    w_vecs = [w_v[pl.ds(q * _L, _L)] for q in range(2 * _EMBED // _L)]
    bias = b_v[pl.ds(0, _L)][0]

    def group(g, carry):
        row_ids = lax.iota(jnp.int32, _L) + g * _L
        acc = jnp.full((_L,), bias, dtype=jnp.float32)
        for j in range(_EMBED):
            col_j = jnp.full((_L,), j, dtype=jnp.int32)
            ucol = plsc.load_gather(u_rows, [row_ids, col_j])
            icol = plsc.load_gather(i_rows, [row_ids, col_j])
            wu = w_vecs[j // _L][j % _L]
            wi = w_vecs[(_EMBED + j) // _L][j % _L]
            acc = acc + ucol * wu + icol * wi
        sig = 1.0 / (1.0 + jnp.exp(-acc))
        out_v[pl.ds(g * _L, _L)] = sig
        return carry

    lax.fori_loop(0, _GROUPS, group, 0)

    pltpu.sync_copy(out_v, out_hbm.at[pl.ds(base, _CHUNK)])


@jax.jit
def _recommender_sc(user_idx, item_idx, user_table, item_table, w_flat, b_pad):
    mesh = plsc.VectorSubcoreMesh(
        core_axis_name="c", subcore_axis_name="s",
        num_cores=_NC, num_subcores=_NS)
    run = functools.partial(
        pl.kernel,
        out_type=jax.ShapeDtypeStruct((_BATCH,), jnp.float32),
        mesh=mesh,
        scratch_types=[
            pltpu.VMEM((_NIDX, _IDXC), jnp.int32),
            pltpu.VMEM((_NIDX, _IDXC), jnp.int32),
            pltpu.VMEM((_CHUNK, _EMBED), jnp.float32),
            pltpu.VMEM((_CHUNK, _EMBED), jnp.float32),
            pltpu.VMEM((2 * _EMBED,), jnp.float32),
            pltpu.VMEM((_L,), jnp.float32),
            pltpu.VMEM((_CHUNK,), jnp.float32),
            pltpu.SemaphoreType.DMA,
        ],
        compiler_params=pltpu.CompilerParams(
            needs_layout_passes=False, use_tc_tiling_on_sc=False),
    )(_sc_body)
    return run(user_idx, item_idx, user_table, item_table, w_flat, b_pad)


def kernel(user_input, item_input, user_table, item_table, W, b):
    ui = user_input.astype(jnp.int32).reshape(_NW * _NIDX, _IDXC)
    ii = item_input.astype(jnp.int32).reshape(_NW * _NIDX, _IDXC)
    w_flat = W.reshape(2 * _EMBED)
    b_pad = jnp.broadcast_to(b.reshape(1), (_L,))
    out = _recommender_sc(ui, ii, user_table, item_table, w_flat, b_pad)
    return out.reshape(_BATCH, 1)


# speedup vs baseline: 6.8626x; 6.8626x over previous
import functools

import jax
import jax.numpy as jnp
from jax import lax
from jax.experimental import pallas as pl
from jax.experimental.pallas import tpu as pltpu
from jax.experimental.pallas import tpu_sc as plsc

_BATCH = 16384
_EMBED = 32
_NC = 2
_NS = 16
_NW = _NC * _NS
_CHUNK = _BATCH // _NW
_L = 16
_IDXC = 128
_NIDX = _CHUNK // _IDXC
_GROUPS = _CHUNK // _L

_PROJ_BLOCK = 32768


def _proj_body(x_ref, w_ref, o_ref):
    x = x_ref[...]
    w = w_ref[...]
    o_ref[...] = jnp.sum(x * w, axis=0)


def _project(table_t, w_col, n_pad):
    n = table_t.shape[1]
    grid = (n_pad + _PROJ_BLOCK - 1) // _PROJ_BLOCK
    return pl.pallas_call(
        _proj_body,
        grid=(grid,),
        in_specs=[
            pl.BlockSpec((_EMBED, _PROJ_BLOCK), lambda i: (0, i)),
            pl.BlockSpec((_EMBED, 1), lambda i: (0, 0)),
        ],
        out_specs=pl.BlockSpec((_PROJ_BLOCK,), lambda i: (i,)),
        out_shape=jax.ShapeDtypeStruct((n_pad,), jnp.float32),
    )(table_t, w_col)


def _sc_body(user_idx_hbm, item_idx_hbm, pu_hbm, pi_hbm, b_hbm, out_hbm,
             idx_u, idx_i, pu_v, pi_v, b_v, out_v, sem):
    wid = lax.axis_index("s") * _NC + lax.axis_index("c")
    base = wid * _CHUNK

    pltpu.sync_copy(user_idx_hbm.at[pl.ds(wid * _NIDX, _NIDX)], idx_u)
    pltpu.sync_copy(item_idx_hbm.at[pl.ds(wid * _NIDX, _NIDX)], idx_i)
    pltpu.sync_copy(b_hbm, b_v)

    copies = []
    for k in range(_NIDX):
        copies.append(pltpu.async_copy(
            pu_hbm.at[idx_u.at[k]], pu_v.at[pl.ds(k * _IDXC, _IDXC)], sem))
        copies.append(pltpu.async_copy(
            pi_hbm.at[idx_i.at[k]], pi_v.at[pl.ds(k * _IDXC, _IDXC)], sem))
    for c in copies:
        c.wait()

    bias = b_v[pl.ds(0, _L)][0]

    def group(g, carry):
        z = pu_v[pl.ds(g * _L, _L)] + pi_v[pl.ds(g * _L, _L)] + bias
        out_v[pl.ds(g * _L, _L)] = 1.0 / (1.0 + jnp.exp(-z))
        return carry

    lax.fori_loop(0, _GROUPS, group, 0)

    pltpu.sync_copy(out_v, out_hbm.at[pl.ds(base, _CHUNK)])


@jax.jit
def _recommender(user_idx, item_idx, user_table, item_table, W, b):
    w_u = W[:_EMBED]
    w_i = W[_EMBED:]
    p_u = _project(user_table.T, w_u, 1000448)
    p_i = _project(item_table.T, w_i, 100352)
    b_pad = jnp.broadcast_to(b.reshape(1), (_L,))

    mesh = plsc.VectorSubcoreMesh(
        core_axis_name="c", subcore_axis_name="s",
        num_cores=_NC, num_subcores=_NS)
    run = functools.partial(
        pl.kernel,
        out_type=jax.ShapeDtypeStruct((_BATCH,), jnp.float32),
        mesh=mesh,
        scratch_types=[
            pltpu.VMEM((_NIDX, _IDXC), jnp.int32),
            pltpu.VMEM((_NIDX, _IDXC), jnp.int32),
            pltpu.VMEM((_CHUNK,), jnp.float32),
            pltpu.VMEM((_CHUNK,), jnp.float32),
            pltpu.VMEM((_L,), jnp.float32),
            pltpu.VMEM((_CHUNK,), jnp.float32),
            pltpu.SemaphoreType.DMA,
        ],
        compiler_params=pltpu.CompilerParams(
            needs_layout_passes=False, use_tc_tiling_on_sc=False),
    )(_sc_body)
    return run(user_idx, item_idx, p_u, p_i, b_pad)


def kernel(user_input, item_input, user_table, item_table, W, b):
    ui = user_input.astype(jnp.int32).reshape(_NW * _NIDX, _IDXC)
    ii = item_input.astype(jnp.int32).reshape(_NW * _NIDX, _IDXC)
    out = _recommender(ui, ii, user_table, item_table, W, b)
    return out.reshape(_BATCH, 1)
